# baseline (speedup 1.0000x reference)
import functools

import jax
import jax.numpy as jnp
from jax import lax
from jax.experimental import pallas as pl
from jax.experimental.pallas import tpu as pltpu, tpu_sc as plsc

BATCH = 16384
HIST = 200
VOCAB = 1000
NSPLIT = 4
BATCH_P = BATCH // NSPLIT
NW = 32
ROWS_W = BATCH_P // NW
R = 64
CHUNK = R * HIST
NCHUNK = ROWS_W // R
L = 16


def _gather_body(items_hbm, pop_hbm, out_hbm, table_v, idx_v, val_v,
                 in_sems, out_sems):
    wid = lax.axis_index("s") * 2 + lax.axis_index("c")
    cbase = wid * NCHUNK
    items_c = items_hbm.reshape(NW * NCHUNK, R, HIST)
    out_c = out_hbm.reshape(NW * NCHUNK, R, HIST)

    pltpu.sync_copy(pop_hbm, table_v)

    def in_copy(c, b):
        return pltpu.make_async_copy(
            items_c.at[cbase + c], idx_v.at[b], in_sems[b])

    def out_copy(c, b):
        return pltpu.make_async_copy(
            val_v.at[b], out_c.at[cbase + c], out_sems[b])

    iota = lax.iota(jnp.int32, L)
    NG = (HIST + L - 1) // L
    cols = [iota + j * L for j in range(NG)]
    masks = [cols[j] < HIST for j in range(NG)]

    in_copy(0, 0).start()
    for c in range(NCHUNK):
        b = c % 2
        if c + 1 < NCHUNK:
            in_copy(c + 1, 1 - b).start()
        in_copy(c, b).wait()
        if c >= 2:
            out_copy(c - 2, b).wait()

        idx_b = idx_v.at[b]
        val_b = val_v.at[b]

        def body(r, _):
            row = jnp.full((L,), 0, jnp.int32) + r
            idxs = [plsc.load_gather(idx_b, [row, cols[j]], mask=masks[j])
                    for j in range(NG)]
            vals = [plsc.load_gather(table_v, [ix], mask=masks[j])
                    for j, ix in enumerate(idxs)]
            for j in range(NG):
                plsc.store_scatter(val_b, [row, cols[j]], vals[j],
                                   mask=masks[j])
            return ()

        lax.fori_loop(0, R, body, ())

        out_copy(c, b).start()

    if NCHUNK >= 2:
        out_copy(NCHUNK - 2, NCHUNK % 2).wait()
    out_copy(NCHUNK - 1, (NCHUNK - 1) % 2).wait()


@jax.jit
def _run(items, popularity):
    mesh = plsc.VectorSubcoreMesh(core_axis_name="c", subcore_axis_name="s")
    k = pl.kernel(
        _gather_body,
        out_type=jax.ShapeDtypeStruct((BATCH_P, HIST), jnp.float32),
        mesh=mesh,
        scratch_types=[
            pltpu.VMEM((VOCAB,), jnp.float32),
            pltpu.VMEM((2, R, HIST), jnp.int32),
            pltpu.VMEM((2, R, HIST), jnp.float32),
            [pltpu.SemaphoreType.DMA, pltpu.SemaphoreType.DMA],
            [pltpu.SemaphoreType.DMA, pltpu.SemaphoreType.DMA],
        ],
        compiler_params=pltpu.CompilerParams(needs_layout_passes=False),
    )
    parts = [k(lax.slice_in_dim(items, s * BATCH_P, (s + 1) * BATCH_P),
               popularity)
             for s in range(NSPLIT)]
    return jnp.concatenate(parts, axis=0)


def kernel(users, items, popularity):
    del users
    return _run(items, popularity)


# speedup vs baseline: 1.6397x; 1.6397x over previous
import functools

import jax
import jax.numpy as jnp
from jax import lax
from jax.experimental import pallas as pl
from jax.experimental.pallas import tpu as pltpu, tpu_sc as plsc

BATCH = 16384
HIST = 200
VOCAB = 1000
NW = 32
ROWS_W = BATCH // NW
R = 64
CHUNK = R * HIST
NCHUNK = ROWS_W // R
L = 16
K = 16


def _gather_body(items_hbm, pop_hbm, out_hbm, table_v, idx_v, val_v,
                 in_sems, out_sems):
    wid = lax.axis_index("s") * 2 + lax.axis_index("c")
    cbase = wid * NCHUNK
    items_c = items_hbm.reshape(NW * NCHUNK, R, HIST)
    out_c = out_hbm.reshape(NW * NCHUNK, R, HIST)

    pltpu.sync_copy(pop_hbm, table_v)

    def in_copy(c, b):
        return pltpu.make_async_copy(
            items_c.at[cbase + c], idx_v.at[b], in_sems[b])

    def out_copy(c, b):
        return pltpu.make_async_copy(
            val_v.at[b], out_c.at[cbase + c], out_sems[b])

    iota = lax.iota(jnp.int32, L)
    NG = (HIST + L - 1) // L
    cols = [iota + j * L for j in range(NG)]
    masks = [cols[j] < HIST for j in range(NG)]

    in_copy(0, 0).start()
    for c in range(NCHUNK):
        b = c % 2
        if c + 1 < NCHUNK:
            in_copy(c + 1, 1 - b).start()
        in_copy(c, b).wait()
        if c >= 2:
            out_copy(c - 2, b).wait()

        idx_b = idx_v.at[b]
        val_b = val_v.at[b]

        def body(r, _):
            for dr in range(2):
                row = jnp.full((L,), 0, jnp.int32) + (2 * r + dr)
                idxs = [plsc.load_gather(idx_b, [row, cols[j]],
                                         mask=masks[j])
                        for j in range(NG)]
                vals = [plsc.load_gather(table_v, [ix], mask=masks[j])
                        for j, ix in enumerate(idxs)]
                for j in range(NG):
                    plsc.store_scatter(val_b, [row, cols[j]], vals[j],
                                       mask=masks[j])
            return ()

        lax.fori_loop(0, R // 2, body, ())

        out_copy(c, b).start()

    out_copy(NCHUNK - 2, NCHUNK % 2).wait()
    out_copy(NCHUNK - 1, (NCHUNK - 1) % 2).wait()


@jax.jit
def _run(items, popularity):
    mesh = plsc.VectorSubcoreMesh(core_axis_name="c", subcore_axis_name="s")
    k = pl.kernel(
        _gather_body,
        out_type=jax.ShapeDtypeStruct((BATCH, HIST), jnp.float32),
        mesh=mesh,
        scratch_types=[
            pltpu.VMEM((VOCAB,), jnp.float32),
            pltpu.VMEM((2, R, HIST), jnp.int32),
            pltpu.VMEM((2, R, HIST), jnp.float32),
            [pltpu.SemaphoreType.DMA, pltpu.SemaphoreType.DMA],
            [pltpu.SemaphoreType.DMA, pltpu.SemaphoreType.DMA],
        ],
        compiler_params=pltpu.CompilerParams(needs_layout_passes=False),
    )
    return k(items, popularity)


def kernel(users, items, popularity):
    del users
    return _run(items, popularity)


# speedup vs baseline: 1.6557x; 1.0098x over previous
import functools

import jax
import jax.numpy as jnp
from jax import lax
from jax.experimental import pallas as pl
from jax.experimental.pallas import tpu as pltpu, tpu_sc as plsc

BATCH = 16384
HIST = 200
VOCAB = 1000
NW = 32
ROWS_W = BATCH // NW
R = 64
CHUNK = R * HIST
NCHUNK = ROWS_W // R
L = 16
K = 16


def _gather_body(items_hbm, pop_hbm, out_hbm, table_v, idx_v, val_v,
                 in_sems, out_sems):
    wid = lax.axis_index("s") * 2 + lax.axis_index("c")
    cbase = wid * NCHUNK
    items_c = items_hbm.reshape(NW * NCHUNK, R, HIST)
    out_c = out_hbm.reshape(NW * NCHUNK, R, HIST)

    pltpu.sync_copy(pop_hbm, table_v)

    def in_copy(c, b):
        return pltpu.make_async_copy(
            items_c.at[cbase + c], idx_v.at[b], in_sems[b])

    def out_copy(c, b):
        return pltpu.make_async_copy(
            val_v.at[b], out_c.at[cbase + c], out_sems[b])

    iota = lax.iota(jnp.int32, L)
    NG = (HIST + L - 1) // L
    cols = [iota + j * L for j in range(NG)]
    masks = [cols[j] < HIST for j in range(NG)]

    in_copy(0, 0).start()
    for c in range(NCHUNK):
        b = c % 2
        if c + 1 < NCHUNK:
            in_copy(c + 1, 1 - b).start()
        in_copy(c, b).wait()
        if c >= 2:
            out_copy(c - 2, b).wait()

        idx_b = idx_v.at[b]
        val_b = val_v.at[b]

        def body(r, _):
            row = jnp.full((L,), 0, jnp.int32) + r
            idxs = [plsc.load_gather(idx_b, [row, cols[j]], mask=masks[j])
                    for j in range(NG)]
            vals = [plsc.load_gather(table_v, [ix], mask=masks[j])
                    for j, ix in enumerate(idxs)]
            for j in range(NG):
                plsc.store_scatter(val_b, [row, cols[j]], vals[j],
                                   mask=masks[j])
            return ()

        lax.fori_loop(0, R, body, ())

        out_copy(c, b).start()

    out_copy(NCHUNK - 2, NCHUNK % 2).wait()
    out_copy(NCHUNK - 1, (NCHUNK - 1) % 2).wait()


@jax.jit
def _run(items, popularity):
    mesh = plsc.VectorSubcoreMesh(core_axis_name="c", subcore_axis_name="s")
    k = pl.kernel(
        _gather_body,
        out_type=jax.ShapeDtypeStruct((BATCH, HIST), jnp.float32),
        mesh=mesh,
        scratch_types=[
            pltpu.VMEM((VOCAB,), jnp.float32),
            pltpu.VMEM((2, R, HIST), jnp.int32),
            pltpu.VMEM((2, R, HIST), jnp.float32),
            [pltpu.SemaphoreType.DMA, pltpu.SemaphoreType.DMA],
            [pltpu.SemaphoreType.DMA, pltpu.SemaphoreType.DMA],
        ],
        compiler_params=pltpu.CompilerParams(needs_layout_passes=False),
    )
    return k(items, popularity)


def kernel(users, items, popularity):
    del users
    return _run(items, popularity)
